# Initial kernel scaffold; baseline (speedup 1.0000x reference)
import jax
import jax.numpy as jnp
from jax.experimental import pallas as pl


def kernel(inputs):
    raise NotImplementedError("write your pallas kernel here")


# speedup vs baseline: 4.0527x; 4.0527x over previous
import functools

import jax
import jax.numpy as jnp
from jax import lax
from jax.experimental import pallas as pl
from jax.experimental.pallas import tpu as pltpu
from jax.experimental.pallas import tpu_sc as plsc

B = 1024
SEQ = 200
CH = 32
GRAM = 3
OUT_N = SEQ + GRAM - 1
IN_ROW = SEQ * CH
SEG = OUT_N * CH
OUT_ROW = GRAM * SEG
PAD = (GRAM - 1) * CH

_info = plsc.get_sparse_core_info()
_NC = _info.num_cores
_NS = _info.num_subcores
_NW = _NC * _NS
_ROWS_PER_W = B // _NW
_RB = 8
_NCHUNK = _ROWS_PER_W // _RB

_COPY_OFF = [j * SEG + (GRAM - 1 - j) * CH for j in range(GRAM)]
_ZERO_REGIONS = [
    (0, PAD),
    (SEG - 32, 32),
    (2 * SEG - 32, 32),
    (GRAM * SEG - PAD, PAD),
]


def _body(in_hbm, out_hbm, in_v, zero_v):
    wid = lax.axis_index("s") * _NC + lax.axis_index("c")
    base0 = wid * _ROWS_PER_W
    zvec = jnp.zeros((16,), jnp.float32)
    for k in range(0, PAD, 16):
        zero_v[pl.ds(k, 16)] = zvec
    for c in range(_NCHUNK):
        base = base0 + c * _RB
        pltpu.sync_copy(in_hbm.at[pl.ds(base * IN_ROW, _RB * IN_ROW)], in_v)
        for r in range(_RB):
            orow = (base + r) * OUT_ROW
            src = in_v.at[pl.ds(r * IN_ROW, IN_ROW)]
            for j in range(GRAM):
                pltpu.sync_copy(src, out_hbm.at[pl.ds(orow + _COPY_OFF[j], IN_ROW)])
            for (off, width) in _ZERO_REGIONS:
                pltpu.sync_copy(zero_v.at[pl.ds(0, width)],
                                out_hbm.at[pl.ds(orow + off, width)])


_ngram_sc = functools.partial(
    pl.kernel,
    out_type=jax.ShapeDtypeStruct((B * OUT_ROW,), jnp.float32),
    mesh=plsc.VectorSubcoreMesh(core_axis_name="c", subcore_axis_name="s"),
    scratch_types=[
        pltpu.VMEM((_RB * IN_ROW,), jnp.float32),
        pltpu.VMEM((PAD,), jnp.float32),
    ],
)(_body)


def kernel(inputs):
    x = inputs.reshape(B * IN_ROW)
    out = _ngram_sc(x)
    return out.reshape(B, GRAM, OUT_N, CH)


# speedup vs baseline: 8.3832x; 2.0686x over previous
import functools

import jax
import jax.numpy as jnp
from jax import lax
from jax.experimental import pallas as pl
from jax.experimental.pallas import tpu as pltpu
from jax.experimental.pallas import tpu_sc as plsc

B = 1024
SEQ = 200
CH = 32
GRAM = 3
OUT_N = SEQ + GRAM - 1
IN_ROW = SEQ * CH
SEG = OUT_N * CH
OUT_ROW = GRAM * SEG
PAD = (GRAM - 1) * CH

_info = plsc.get_sparse_core_info()
_NC = _info.num_cores
_NS = _info.num_subcores
_NW = _NC * _NS
_ROWS_PER_W = B // _NW
_RB = 8
_NCHUNK = _ROWS_PER_W // _RB

_COPY_OFF = [j * SEG + (GRAM - 1 - j) * CH for j in range(GRAM)]
_ZERO_REGIONS = [
    (0, PAD),
    (SEG - 32, 32),
    (2 * SEG - 32, 32),
    (GRAM * SEG - PAD, PAD),
]


def _body(in_hbm, out_hbm, in_v, zero_v):
    wid = lax.axis_index("s") * _NC + lax.axis_index("c")
    base0 = wid * _ROWS_PER_W
    zvec = jnp.zeros((16,), jnp.float32)
    for r in range(_RB):
        for k in range(0, PAD, 16):
            zero_v[r, pl.ds(k, 16)] = zvec
    for c in range(_NCHUNK):
        rows = pl.ds(base0 + c * _RB, _RB)
        pltpu.sync_copy(in_hbm.at[rows], in_v)
        for j in range(GRAM):
            pltpu.sync_copy(in_v, out_hbm.at[rows, pl.ds(_COPY_OFF[j], IN_ROW)])
        for (off, width) in _ZERO_REGIONS:
            pltpu.sync_copy(zero_v.at[:, pl.ds(0, width)],
                            out_hbm.at[rows, pl.ds(off, width)])


_ngram_sc = functools.partial(
    pl.kernel,
    out_type=jax.ShapeDtypeStruct((B, OUT_ROW), jnp.float32),
    mesh=plsc.VectorSubcoreMesh(core_axis_name="c", subcore_axis_name="s"),
    scratch_types=[
        pltpu.VMEM((_RB, IN_ROW), jnp.float32),
        pltpu.VMEM((_RB, PAD), jnp.float32),
    ],
    compiler_params=pltpu.CompilerParams(use_tc_tiling_on_sc=False),
)(_body)


def kernel(inputs):
    x = inputs.reshape(B, IN_ROW)
    out = _ngram_sc(x)
    return out.reshape(B, GRAM, OUT_N, CH)


# speedup vs baseline: 8.4432x; 1.0072x over previous
import functools

import jax
import jax.numpy as jnp
from jax import lax
from jax.experimental import pallas as pl
from jax.experimental.pallas import tpu as pltpu
from jax.experimental.pallas import tpu_sc as plsc

B = 1024
SEQ = 200
CH = 32
GRAM = 3
OUT_N = SEQ + GRAM - 1
IN_ROW = SEQ * CH
SEG = OUT_N * CH
OUT_ROW = GRAM * SEG
PAD = (GRAM - 1) * CH

_info = plsc.get_sparse_core_info()
_NC = _info.num_cores
_NS = _info.num_subcores
_NW = _NC * _NS
_ROWS_PER_W = B // _NW
_RB = 8
_NCHUNK = _ROWS_PER_W // _RB

_COPY_OFF = [j * SEG + (GRAM - 1 - j) * CH for j in range(GRAM)]
_ZERO_REGIONS = [
    (0, PAD),
    (SEG - 32, 32),
    (2 * SEG - 32, 32),
    (GRAM * SEG - PAD, PAD),
]


def _body(in_hbm, out_hbm, in_v0, in_v1, zero_v, rsems, wsems, zsems):
    wid = lax.axis_index("s") * _NC + lax.axis_index("c")
    base0 = wid * _ROWS_PER_W
    bufs = (in_v0, in_v1)

    zvec = jnp.zeros((16,), jnp.float32)
    for r in range(_ROWS_PER_W):
        for k in range(0, PAD, 16):
            zero_v[r, pl.ds(k, 16)] = zvec
    all_rows = pl.ds(base0, _ROWS_PER_W)
    zw = [pltpu.async_copy(zero_v.at[:, pl.ds(0, width)],
                           out_hbm.at[all_rows, pl.ds(off, width)],
                           zsems.at[i])
          for i, (off, width) in enumerate(_ZERO_REGIONS)]

    def rows(c):
        return pl.ds(base0 + c * _RB, _RB)

    reads = [None] * _NCHUNK
    writes = [[] for _ in range(_NCHUNK)]
    reads[0] = pltpu.async_copy(in_hbm.at[rows(0)], bufs[0], rsems.at[0])
    for c in range(_NCHUNK):
        reads[c].wait()
        buf = bufs[c % 2]
        for j in range(GRAM):
            writes[c].append(pltpu.async_copy(
                buf, out_hbm.at[rows(c), pl.ds(_COPY_OFF[j], IN_ROW)],
                wsems.at[c % 2, j]))
        if c + 1 < _NCHUNK:
            if c >= 1:
                for h in writes[c - 1]:
                    h.wait()
            reads[c + 1] = pltpu.async_copy(
                in_hbm.at[rows(c + 1)], bufs[(c + 1) % 2],
                rsems.at[(c + 1) % 2])
    for c in (_NCHUNK - 2, _NCHUNK - 1):
        for h in writes[c]:
            h.wait()
    for h in zw:
        h.wait()


_ngram_sc = functools.partial(
    pl.kernel,
    out_type=jax.ShapeDtypeStruct((B, OUT_ROW), jnp.float32),
    mesh=plsc.VectorSubcoreMesh(core_axis_name="c", subcore_axis_name="s"),
    scratch_types=[
        pltpu.VMEM((_RB, IN_ROW), jnp.float32),
        pltpu.VMEM((_RB, IN_ROW), jnp.float32),
        pltpu.VMEM((_ROWS_PER_W, PAD), jnp.float32),
        pltpu.SemaphoreType.DMA((2,)),
        pltpu.SemaphoreType.DMA((2, GRAM)),
        pltpu.SemaphoreType.DMA((len(_ZERO_REGIONS),)),
    ],
    compiler_params=pltpu.CompilerParams(use_tc_tiling_on_sc=False),
)(_body)


def kernel(inputs):
    x = inputs.reshape(B, IN_ROW)
    out = _ngram_sc(x)
    return out.reshape(B, GRAM, OUT_N, CH)


# speedup vs baseline: 31.3742x; 3.7159x over previous
import functools

import jax
import jax.numpy as jnp
from jax import lax
from jax.experimental import pallas as pl
from jax.experimental.pallas import tpu as pltpu
from jax.experimental.pallas import tpu_sc as plsc

B = 1024
SEQ = 200
CH = 32
GRAM = 3
OUT_N = SEQ + GRAM - 1

_info = plsc.get_sparse_core_info()
_NC = _info.num_cores
_NS = _info.num_subcores
_NW = _NC * _NS

_N7 = SEQ - 6 * _NW
_BLK7, _BLK6 = 7, 6


def _copy_blocks(xt, yt, bufs, rsems, wsems, t_start, nblk):
    reads = [None] * nblk
    writes = [[] for _ in range(nblk)]
    reads[0] = pltpu.async_copy(xt.at[pl.ds(t_start, 1)], bufs[0], rsems.at[0])
    for k in range(nblk):
        reads[k].wait()
        buf = bufs[k % 2]
        for j in range(GRAM):
            writes[k].append(pltpu.async_copy(
                buf, yt.at[j, pl.ds(t_start + k + (GRAM - 1 - j), 1)],
                wsems.at[k % 2, j]))
        if k + 1 < nblk:
            if k >= 1:
                for h in writes[k - 1]:
                    h.wait()
            reads[k + 1] = pltpu.async_copy(
                xt.at[pl.ds(t_start + k + 1, 1)], bufs[(k + 1) % 2],
                rsems.at[(k + 1) % 2])
    for kk in (nblk - 2, nblk - 1):
        for h in writes[kk]:
            h.wait()


def _body(xt, yt, buf0, buf1, zero_v, rsems, wsems, zsem):
    wid = lax.axis_index("s") * _NC + lax.axis_index("c")
    bufs = (buf0, buf1)

    @pl.when(wid < _N7)
    def _():
        _copy_blocks(xt, yt, bufs, rsems, wsems, _BLK7 * wid, _BLK7)

    @pl.when(wid >= _N7)
    def _():
        _copy_blocks(xt, yt, bufs, rsems, wsems,
                     _BLK6 * wid + _N7, _BLK6)

    @pl.when((wid >= _N7) & (wid < _N7 + 2 * GRAM))
    def _():
        z16 = jnp.zeros((16,), jnp.float32)

        def zinit(i, carry):
            r = i // (B // 16)
            k = (i % (B // 16)) * 16
            zero_v[0, r, pl.ds(k, 16)] = z16
            return carry

        lax.fori_loop(0, CH * (B // 16), zinit, 0)
        z = wid - _N7
        jz = z // 2
        iz = jnp.where(z % 2 == 0, SEQ * (z // 4),
                       1 + SEQ * jnp.int32(z >= GRAM))
        pltpu.async_copy(zero_v, yt.at[jz, pl.ds(iz, 1)], zsem).wait()


_ngram_sc = functools.partial(
    pl.kernel,
    out_type=jax.ShapeDtypeStruct((GRAM, OUT_N, CH, B), jnp.float32),
    mesh=plsc.VectorSubcoreMesh(core_axis_name="c", subcore_axis_name="s"),
    scratch_types=[
        pltpu.VMEM((1, CH, B), jnp.float32),
        pltpu.VMEM((1, CH, B), jnp.float32),
        pltpu.VMEM((1, CH, B), jnp.float32),
        pltpu.SemaphoreType.DMA((2,)),
        pltpu.SemaphoreType.DMA((2, GRAM)),
        pltpu.SemaphoreType.DMA,
    ],
)(_body)


def kernel(inputs):
    xt = jnp.transpose(inputs, (1, 2, 0))
    yt = _ngram_sc(xt)
    return jnp.transpose(yt, (3, 0, 1, 2))


# speedup vs baseline: 34.5072x; 1.0999x over previous
import functools

import jax
import jax.numpy as jnp
from jax import lax
from jax.experimental import pallas as pl
from jax.experimental.pallas import tpu as pltpu
from jax.experimental.pallas import tpu_sc as plsc

B = 1024
SEQ = 200
CH = 32
GRAM = 3
OUT_N = SEQ + GRAM - 1

_info = plsc.get_sparse_core_info()
_NC = _info.num_cores
_NS = _info.num_subcores
_NW = _NC * _NS

_N7 = SEQ - 6 * _NW
_BLK7, _BLK6 = 7, 6


def _copy_blocks(xt, yt, bufs, rsems, wsems, t_start, nblk):
    nb = len(bufs)
    reads = [None] * nblk
    writes = [[] for _ in range(nblk)]
    for p in range(min(nb - 1, nblk)):
        reads[p] = pltpu.async_copy(
            xt.at[pl.ds(t_start + p, 1)], bufs[p % nb], rsems.at[p % nb])
    for k in range(nblk):
        reads[k].wait()
        buf = bufs[k % nb]
        for j in range(GRAM):
            writes[k].append(pltpu.async_copy(
                buf, yt.at[j, pl.ds(t_start + k + (GRAM - 1 - j), 1)],
                wsems.at[k % nb, j]))
        nxt = k + nb - 1
        if nxt < nblk:
            if k >= 1:
                for h in writes[k - 1]:
                    h.wait()
            reads[nxt] = pltpu.async_copy(
                xt.at[pl.ds(t_start + nxt, 1)], bufs[nxt % nb],
                rsems.at[nxt % nb])
    for kk in range(max(0, nblk - 2), nblk):
        for h in writes[kk]:
            h.wait()


def _body(xt, yt, buf0, buf1, buf2, zero_v, rsems, wsems, zsems):
    wid = lax.axis_index("s") * _NC + lax.axis_index("c")
    bufs = (buf0, buf1, buf2)

    @pl.when(wid < _N7)
    def _():
        _copy_blocks(xt, yt, bufs, rsems, wsems, _BLK7 * wid, _BLK7)

    @pl.when(wid >= _N7)
    def _():
        _copy_blocks(xt, yt, bufs, rsems, wsems,
                     _BLK6 * wid + _N7, _BLK6)

    @pl.when((wid >= _N7) & (wid < _N7 + 2 * GRAM))
    def _():
        z16 = jnp.zeros((16,), jnp.float32)

        def zinit(i, carry):
            r = i // (B // 16)
            k = (i % (B // 16)) * 16
            zero_v[0, r, pl.ds(k, 16)] = z16
            return carry

        lax.fori_loop(0, 8 * (B // 16), zinit, 0)
        z = wid - _N7
        jz = z // 2
        iz = jnp.where(z % 2 == 0, SEQ * (z // 4),
                       1 + SEQ * jnp.int32(z >= GRAM))
        zw = [pltpu.async_copy(zero_v,
                               yt.at[jz, pl.ds(iz, 1), pl.ds(c, 8)],
                               zsems.at[c // 8])
              for c in range(0, CH, 8)]
        for h in zw:
            h.wait()


_ngram_sc = functools.partial(
    pl.kernel,
    out_type=jax.ShapeDtypeStruct((GRAM, OUT_N, CH, B), jnp.float32),
    mesh=plsc.VectorSubcoreMesh(core_axis_name="c", subcore_axis_name="s"),
    scratch_types=[
        pltpu.VMEM((1, CH, B), jnp.float32),
        pltpu.VMEM((1, CH, B), jnp.float32),
        pltpu.VMEM((1, CH, B), jnp.float32),
        pltpu.VMEM((1, 8, B), jnp.float32),
        pltpu.SemaphoreType.DMA((3,)),
        pltpu.SemaphoreType.DMA((3, GRAM)),
        pltpu.SemaphoreType.DMA((CH // 8,)),
    ],
)(_body)


def kernel(inputs):
    xt = jnp.transpose(inputs, (1, 2, 0))
    yt = _ngram_sc(xt)
    return jnp.transpose(yt, (3, 0, 1, 2))


# speedup vs baseline: 34.9185x; 1.0119x over previous
import functools

import jax
import jax.numpy as jnp
from jax import lax
from jax.experimental import pallas as pl
from jax.experimental.pallas import tpu as pltpu
from jax.experimental.pallas import tpu_sc as plsc

B = 1024
SEQ = 200
CH = 32
GRAM = 3
OUT_N = SEQ + GRAM - 1
HCH = CH // 2
NHALF = 2 * SEQ

_info = plsc.get_sparse_core_info()
_NC = _info.num_cores
_NS = _info.num_subcores
_NW = _NC * _NS

_N13 = NHALF - 12 * _NW
_NB = 6
_PD = 3
_ZW0 = _NW - 2 * GRAM


def _copy_halves(xt, yt, bufs, rsems, wsems, h_start, nh):
    reads = [None] * nh
    writes = [[] for _ in range(nh)]
    waited = set()

    def issue_read(h):
        g = h_start + h
        return pltpu.async_copy(
            xt.at[pl.ds(g // 2, 1), pl.ds((g % 2) * HCH, HCH)],
            bufs[h % _NB], rsems.at[h % _NB])

    for p in range(min(_PD, nh)):
        reads[p] = issue_read(p)
    for k in range(nh):
        reads[k].wait()
        g = h_start + k
        t, co = g // 2, (g % 2) * HCH
        for j in range(GRAM):
            writes[k].append(pltpu.async_copy(
                bufs[k % _NB],
                yt.at[j, pl.ds(t + (GRAM - 1 - j), 1), pl.ds(co, HCH)],
                wsems.at[k % _NB, j]))
        nxt = k + _PD
        if nxt < nh:
            old = k - (_NB - _PD)
            if old >= 0:
                for h in writes[old]:
                    h.wait()
                waited.add(old)
            reads[nxt] = issue_read(nxt)
    for k in range(nh):
        if k not in waited:
            for h in writes[k]:
                h.wait()


def _body(xt, yt, b0, b1, b2, b3, b4, b5, zero_v, rsems, wsems, zsems):
    wid = lax.axis_index("s") * _NC + lax.axis_index("c")
    bufs = (b0, b1, b2, b3, b4, b5)

    @pl.when(wid < _N13)
    def _():
        _copy_halves(xt, yt, bufs, rsems, wsems, 13 * wid, 13)

    @pl.when(wid >= _N13)
    def _():
        _copy_halves(xt, yt, bufs, rsems, wsems, 12 * wid + _N13, 12)

    @pl.when(wid >= _ZW0)
    def _():
        z16 = jnp.zeros((16,), jnp.float32)

        def zinit(i, carry):
            r = i // (B // 16)
            k = (i % (B // 16)) * 16
            zero_v[0, r, pl.ds(k, 16)] = z16
            return carry

        lax.fori_loop(0, 8 * (B // 16), zinit, 0)
        z = wid - _ZW0
        jz = z // 2
        iz = jnp.where(z % 2 == 0, SEQ * (z // 4),
                       1 + SEQ * jnp.int32(z >= GRAM))
        zw = [pltpu.async_copy(zero_v,
                               yt.at[jz, pl.ds(iz, 1), pl.ds(c, 8)],
                               zsems.at[c // 8])
              for c in range(0, CH, 8)]
        for h in zw:
            h.wait()


_ngram_sc = functools.partial(
    pl.kernel,
    out_type=jax.ShapeDtypeStruct((GRAM, OUT_N, CH, B), jnp.float32),
    mesh=plsc.VectorSubcoreMesh(core_axis_name="c", subcore_axis_name="s"),
    scratch_types=(
        [pltpu.VMEM((1, HCH, B), jnp.float32) for _ in range(_NB)]
        + [
            pltpu.VMEM((1, 8, B), jnp.float32),
            pltpu.SemaphoreType.DMA((_NB,)),
            pltpu.SemaphoreType.DMA((_NB, GRAM)),
            pltpu.SemaphoreType.DMA((CH // 8,)),
        ]
    ),
)(_body)


def kernel(inputs):
    xt = jnp.transpose(inputs, (1, 2, 0))
    yt = _ngram_sc(xt)
    return jnp.transpose(yt, (3, 0, 1, 2))
